# baseline (speedup 1.0000x reference)
import functools
import math

import jax
import jax.numpy as jnp
from jax import lax
from jax.experimental import pallas as pl
from jax.experimental.pallas import tpu as pltpu
from jax.experimental.pallas import tpu_sc as plsc

D = 1024
ROWS = 2 * 4096 - 1
N = ROWS * D
BATCH = 4
NC = 2
NS = 16
NW = NC * NS
CHUNK = 32 * D
NCHUNK_TOTAL = (N + CHUNK - 1) // CHUNK
CHUNKS_PER_W = NCHUNK_TOTAL // NW
LANES = 16
UNROLL = 8
SCALE = math.sqrt(D)


def _sc_body(w_hbm, out_hbm, buf, sem_in, sem_out):
    c = lax.axis_index("c")
    s = lax.axis_index("s")
    wid = s * NC + c

    def chunk_body(g, carry):
        k = wid * CHUNKS_PER_W + g
        base = lax.min(k * CHUNK, N - CHUNK)
        pltpu.async_copy(w_hbm.at[pl.ds(base, CHUNK)], buf, sem_in).wait()

        def vec_body(i, rc):
            for j in range(UNROLL):
                sl = pl.ds(i * (LANES * UNROLL) + j * LANES, LANES)
                buf[sl] = buf[sl] * SCALE
            return rc

        lax.fori_loop(0, CHUNK // (LANES * UNROLL), vec_body, 0)

        cps = [
            pltpu.async_copy(buf, out_hbm.at[pl.ds(b * N + base, CHUNK)], sem_out)
            for b in range(BATCH)
        ]
        for cp in cps:
            cp.wait()
        return carry

    lax.fori_loop(0, CHUNKS_PER_W, chunk_body, 0)


def _sc_embed(w_flat):
    mesh = plsc.VectorSubcoreMesh(core_axis_name="c", subcore_axis_name="s")
    f = functools.partial(
        pl.kernel,
        mesh=mesh,
        out_type=jax.ShapeDtypeStruct((BATCH * N,), jnp.float32),
        scratch_types=[
            pltpu.VMEM((CHUNK,), jnp.float32),
            pltpu.SemaphoreType.DMA,
            pltpu.SemaphoreType.DMA,
        ],
    )(_sc_body)
    return f(w_flat)


def kernel(input, weights):
    del input
    out_flat = _sc_embed(weights.reshape(N))
    return out_flat.reshape(BATCH, ROWS, D)


# speedup vs baseline: 1.0128x; 1.0128x over previous
import functools
import math

import jax
import jax.numpy as jnp
from jax import lax
from jax.experimental import pallas as pl
from jax.experimental.pallas import tpu as pltpu
from jax.experimental.pallas import tpu_sc as plsc

D = 1024
ROWS = 2 * 4096 - 1
N = ROWS * D
BATCH = 4
NC = 2
NS = 16
NW = NC * NS
CHUNK = 16 * D
NCHUNK_TOTAL = (N + CHUNK - 1) // CHUNK
CHUNKS_PER_W = NCHUNK_TOTAL // NW
NBUF = 4
LOOKAHEAD = 2
LANES = 16
SCALE = math.sqrt(D)


def _sc_body(w_hbm, out_hbm, b0, b1, b2, b3, si0, si1, si2, si3,
             so0, so1, so2, so3):
    c = lax.axis_index("c")
    s = lax.axis_index("s")
    wid = s * NC + c
    bufs = [b0, b1, b2, b3]
    sem_in = [si0, si1, si2, si3]
    sem_out = [so0, so1, so2, so3]

    def base(k):
        return lax.min((wid * CHUNKS_PER_W + k) * CHUNK, N - CHUNK)

    def issue_gather(k):
        return pltpu.async_copy(
            w_hbm.at[pl.ds(base(k), CHUNK)], bufs[k % NBUF], sem_in[k % NBUF])

    def issue_scatters(k):
        return [
            pltpu.async_copy(
                bufs[k % NBUF],
                out_hbm.at[pl.ds(b * N + base(k), CHUNK)],
                sem_out[k % NBUF])
            for b in range(BATCH)
        ]

    gathers = {k: issue_gather(k) for k in range(LOOKAHEAD)}
    scatters = {}
    for g in range(CHUNKS_PER_W):
        if g - LOOKAHEAD in scatters:
            for h in scatters.pop(g - LOOKAHEAD):
                h.wait()
        if g + LOOKAHEAD < CHUNKS_PER_W:
            gathers[g + LOOKAHEAD] = issue_gather(g + LOOKAHEAD)
        gathers.pop(g).wait()

        buf = bufs[g % NBUF]

        @plsc.parallel_loop(0, CHUNK // LANES, 1, unroll=8)
        def _scale(i):
            sl = pl.ds(i * LANES, LANES)
            buf[sl] = buf[sl] * SCALE

        scatters[g] = issue_scatters(g)

    for hs in scatters.values():
        for h in hs:
            h.wait()


def _sc_embed(w_flat):
    mesh = plsc.VectorSubcoreMesh(core_axis_name="c", subcore_axis_name="s")
    f = functools.partial(
        pl.kernel,
        mesh=mesh,
        out_type=jax.ShapeDtypeStruct((BATCH * N,), jnp.float32),
        scratch_types=(
            [pltpu.VMEM((CHUNK,), jnp.float32) for _ in range(NBUF)]
            + [pltpu.SemaphoreType.DMA for _ in range(2 * NBUF)]
        ),
    )(_sc_body)
    return f(w_flat)


def kernel(input, weights):
    del input
    out_flat = _sc_embed(weights.reshape(N))
    return out_flat.reshape(BATCH, ROWS, D)
